# baseline (speedup 1.0000x reference)
import functools

import jax
import jax.numpy as jnp
from jax import lax
from jax.experimental import pallas as pl
from jax.experimental.pallas import tpu as pltpu
from jax.experimental.pallas import tpu_sc as plsc

B = 16384
D = 64
CTX = 10
NEG = 5
NV = 1 + NEG
ROWS = 1000000
SROWS = ROWS // 2
NC = 2
NS = 16
NW = NC * NS
PERW = B // NW
C = 16
NCHUNK = PERW // C
URPC = C * CTX
VRPC = C * NV

U_GROUPS = [(0, 128), (128, 32)]
V_GROUPS = [(0, 96)]


_mesh = plsc.VectorSubcoreMesh(
    core_axis_name="c", subcore_axis_name="s", num_cores=NC, num_subcores=NS)

_CH = 168
_NCH = 93
_WROWS = _CH * _NCH
_REM = SROWS - NW * _WROWS


@functools.partial(
    pl.kernel,
    out_type=[
        jax.ShapeDtypeStruct((SROWS, 2 * D), jnp.float32),
        jax.ShapeDtypeStruct((SROWS, 2 * D), jnp.float32),
    ],
    mesh=_mesh,
    scratch_types=[
        pltpu.VMEM((2, 2 * _CH, D), jnp.float32),
        pltpu.VMEM((2, _CH, 2 * D), jnp.float32),
        pltpu.SemaphoreType.DMA,
        pltpu.SemaphoreType.DMA,
    ],
    compiler_params=pltpu.CompilerParams(needs_layout_passes=False),
)
def _pack(u_tab, v_tab, up_hbm, vp_hbm, rbuf, wbuf, sem_r, sem_w):
    wid = lax.axis_index("s") * NC + lax.axis_index("c")
    s0 = wid * _WROWS

    def run(tab, out):
        def rd(ch, par):
            pltpu.async_copy(
                tab.at[pl.ds((s0 + ch * _CH) * 2, 2 * _CH), :],
                rbuf.at[par], sem_r)

        def rd_drain(ch, par):
            pltpu.make_async_copy(
                tab.at[pl.ds((s0 + ch * _CH) * 2, 2 * _CH), :],
                rbuf.at[par], sem_r).wait()

        def wr(ch, par):
            pltpu.async_copy(
                wbuf.at[par], out.at[pl.ds(s0 + ch * _CH, _CH)], sem_w)

        def wr_drain(ch, par):
            pltpu.make_async_copy(
                wbuf.at[par], out.at[pl.ds(s0 + ch * _CH, _CH)], sem_w).wait()

        def assemble(par):
            def row(i, _):
                for half in (0, 1):
                    for k in range(D // 16):
                        wbuf[par, i, pl.ds(half * D + k * 16, 16)] = (
                            rbuf[par, 2 * i + half, pl.ds(k * 16, 16)])
                return _
            lax.fori_loop(0, _CH, row, 0, unroll=4)

        def chunk(ch, par, pred_next, pred_prev):
            rd_drain(ch, par)

            @pl.when(pred_next)
            def _n():
                rd(ch + 1, 1 - par)

            @pl.when(pred_prev)
            def _p():
                wr_drain(ch - 2, par)

            assemble(par)
            wr(ch, par)

        rd(0, 0)

        def pair(i, _):
            chunk(2 * i, 0, jnp.bool_(True), i >= 1)
            chunk(2 * i + 1, 1, 2 * i + 2 < _NCH, i >= 1)
            return _

        lax.fori_loop(0, _NCH // 2, pair, 0)
        chunk(_NCH - 1, 0, jnp.bool_(False), jnp.bool_(True))
        wr_drain(_NCH - 2, 1)
        wr_drain(_NCH - 1, 0)

    run(u_tab, up_hbm)
    run(v_tab, vp_hbm)

    @pl.when(wid == NW - 1)
    def _rem():
        base = NW * _WROWS
        for tab, out in ((u_tab, up_hbm), (v_tab, vp_hbm)):
            pltpu.async_copy(
                tab.at[pl.ds(base * 2, 2 * _REM), :],
                rbuf.at[0, pl.ds(0, 2 * _REM)], sem_r).wait()

            def rrow(i, _):
                for half in (0, 1):
                    for k in range(D // 16):
                        wbuf[0, i, pl.ds(half * D + k * 16, 16)] = (
                            rbuf[0, 2 * i + half, pl.ds(k * 16, 16)])
                return _

            lax.fori_loop(0, _REM, rrow, 0, unroll=4)
            pltpu.async_copy(
                wbuf.at[0, pl.ds(0, _REM)],
                out.at[pl.ds(base, _REM)], sem_w).wait()


@functools.partial(
    pl.kernel,
    out_type=jax.ShapeDtypeStruct((B * NV,), jnp.float32),
    mesh=_mesh,
    scratch_types=[
        pltpu.VMEM((PERW * CTX,), jnp.int32),
        pltpu.VMEM((PERW * NV,), jnp.int32),
        pltpu.VMEM((PERW * CTX,), jnp.int32),
        pltpu.VMEM((PERW * NV,), jnp.int32),
        pltpu.VMEM((PERW * CTX,), jnp.int32),
        pltpu.VMEM((PERW * NV,), jnp.int32),
        pltpu.VMEM((2 * URPC, 2 * D), jnp.float32),
        pltpu.VMEM((2 * VRPC, 2 * D), jnp.float32),
        pltpu.VMEM((PERW * NV,), jnp.float32),
        pltpu.SemaphoreType.DMA,
        pltpu.SemaphoreType.DMA,
    ],
    compiler_params=pltpu.CompilerParams(needs_layout_passes=False),
)
def _sc_scores(uidx_hbm, vidx_hbm, u_pack, v_pack, out_hbm,
               uidx_v, vidx_v, usid_v, vsid_v, uh_v, vh_v,
               ubuf, vbuf, scores, sem0, sem1):
    wid = lax.axis_index("s") * NC + lax.axis_index("c")
    base = wid * PERW
    iota = lax.iota(jnp.int32, 16)
    pltpu.sync_copy(uidx_hbm.at[pl.ds(base * CTX, PERW * CTX)], uidx_v)
    pltpu.sync_copy(vidx_hbm.at[pl.ds(base * NV, PERW * NV)], vidx_v)

    def split(g, _, raw, sid, half, n16):
        del n16
        x = raw[pl.ds(g * 16, 16)]
        sid[pl.ds(g * 16, 16)] = x >> 1
        half[pl.ds(g * 16, 16)] = (x & 1) << 6
        return _

    lax.fori_loop(0, PERW * CTX // 16,
                  functools.partial(split, raw=uidx_v, sid=usid_v, half=uh_v,
                                    n16=None), 0)
    lax.fori_loop(0, PERW * NV // 16,
                  functools.partial(split, raw=vidx_v, sid=vsid_v, half=vh_v,
                                    n16=None), 0)

    def issue(ch, par, sem):
        for off, n in U_GROUPS:
            pltpu.async_copy(
                u_pack.at[usid_v.at[pl.ds(ch * URPC + off, n)]],
                ubuf.at[pl.ds(par * URPC + off, n)], sem)
        for off, n in V_GROUPS:
            pltpu.async_copy(
                v_pack.at[vsid_v.at[pl.ds(ch * VRPC + off, n)]],
                vbuf.at[pl.ds(par * VRPC + off, n)], sem)

    def drain(ch, par, sem):
        for off, n in U_GROUPS:
            pltpu.make_async_copy(
                u_pack.at[usid_v.at[pl.ds(ch * URPC + off, n)]],
                ubuf.at[pl.ds(par * URPC + off, n)], sem).wait()
        for off, n in V_GROUPS:
            pltpu.make_async_copy(
                v_pack.at[vsid_v.at[pl.ds(ch * VRPC + off, n)]],
                vbuf.at[pl.ds(par * VRPC + off, n)], sem).wait()

    def compute(ch, par):
        urow = [par * URPC + iota * CTX + c for c in range(CTX)]
        ucol = [plsc.load_gather(uh_v, [ch * URPC + iota * CTX + c])
                for c in range(CTX)]
        vrow = [par * VRPC + iota * NV + t for t in range(NV)]
        vcol = [plsc.load_gather(vh_v, [ch * VRPC + iota * NV + t])
                for t in range(NV)]

        def dstep(d, acc):
            us = plsc.load_gather(ubuf, [urow[0], ucol[0] + d])
            for c in range(1, CTX):
                us = us + plsc.load_gather(ubuf, [urow[c], ucol[c] + d])
            return tuple(
                acc[t] + us * plsc.load_gather(vbuf, [vrow[t], vcol[t] + d])
                for t in range(NV))

        zero = jnp.zeros((16,), jnp.float32)
        acc = lax.fori_loop(0, D, dstep, (zero,) * NV)
        for t in range(NV):
            sgn = 1.0 / CTX if t == 0 else -1.0 / CTX
            scores[pl.ds((ch * NV + t) * 16, 16)] = acc[t] * sgn

    issue(0, 0, sem0)

    def pair(i, _):
        ch0 = i * 2
        drain(ch0, 0, sem0)

        @pl.when(i == 0)
        def _first():
            issue(1, 1, sem1)

        @pl.when(ch0 + 2 < NCHUNK)
        def _next0():
            issue(ch0 + 2, 0, sem0)

        compute(ch0, 0)
        drain(ch0 + 1, 1, sem1)

        @pl.when(ch0 + 3 < NCHUNK)
        def _next1():
            issue(ch0 + 3, 1, sem1)

        compute(ch0 + 1, 1)
        return _

    lax.fori_loop(0, NCHUNK // 2, pair, 0)
    pltpu.sync_copy(scores, out_hbm.at[pl.ds(base * NV, PERW * NV)])


def _loss_body(x_ref, o_ref):
    o_ref[0, 0] = -jnp.sum(jax.nn.log_sigmoid(x_ref[...]))


_loss = pl.pallas_call(
    _loss_body,
    out_shape=jax.ShapeDtypeStruct((1, 1), jnp.float32),
    out_specs=pl.BlockSpec(memory_space=pltpu.SMEM),
)


def kernel(batch_0, batch_1, batch_2, u_table, v_table):
    uidx = batch_0.astype(jnp.int32).reshape(B * CTX)
    vidx = jnp.concatenate(
        [batch_1[:, None], batch_2], axis=1).astype(jnp.int32).reshape(B * NV)
    u_pack, v_pack = _pack(u_table, v_table)
    scores = _sc_scores(uidx, vidx, u_pack, v_pack)
    loss = _loss(scores.reshape(B * NV // 128, 128))
    return loss.reshape(())


# speedup vs baseline: 1.4989x; 1.4989x over previous
import functools

import jax
import jax.numpy as jnp
from jax import lax
from jax.experimental import pallas as pl
from jax.experimental.pallas import tpu as pltpu
from jax.experimental.pallas import tpu_sc as plsc

B = 16384
D = 64
CTX = 10
NEG = 5
NV = 1 + NEG
NC = 2
NS = 16
NW = NC * NS
PERW = B // NW
C = 32
NCHUNK = PERW // C

U_GROUPS = [(0, 128), (128, 128), (256, 64)]
V_GROUPS = [(0, 128), (128, 64)]

_mesh = plsc.VectorSubcoreMesh(
    core_axis_name="c", subcore_axis_name="s", num_cores=NC, num_subcores=NS)

_SC_PARAMS = pltpu.CompilerParams(
    needs_layout_passes=False, use_tc_tiling_on_sc=False)


@functools.partial(
    pl.kernel,
    out_type=jax.ShapeDtypeStruct((B * D,), jnp.float32),
    mesh=_mesh,
    scratch_types=[
        pltpu.VMEM((PERW * CTX,), jnp.int32),
        pltpu.VMEM((C * CTX, D), jnp.float32),
        pltpu.VMEM((PERW * D,), jnp.float32),
        pltpu.SemaphoreType.DMA,
    ],
    compiler_params=_SC_PARAMS,
)
def _sc_pool(uidx_hbm, u_tab, out_hbm, uidx_v, urows, uemb, sem):
    wid = lax.axis_index("s") * NC + lax.axis_index("c")
    base = wid * PERW
    pltpu.sync_copy(uidx_hbm.at[pl.ds(base * CTX, PERW * CTX)], uidx_v)

    def chunk(ch, carry):
        handles = []
        for off, n in U_GROUPS:
            handles.append(pltpu.async_copy(
                u_tab.at[uidx_v.at[pl.ds(ch * (C * CTX) + off, n)]],
                urows.at[pl.ds(off, n)], sem))
        for h in handles:
            h.wait()

        def elem(e, carry2):
            urow0 = e * CTX
            acc = [urows[urow0, pl.ds(k * 16, 16)] for k in range(4)]
            for c in range(1, CTX):
                for k in range(4):
                    acc[k] = acc[k] + urows[urow0 + c, pl.ds(k * 16, 16)]
            ebase = (ch * C + e) * D
            for k in range(4):
                uemb[pl.ds(ebase + k * 16, 16)] = acc[k] * (1.0 / CTX)
            return carry2

        lax.fori_loop(0, C, elem, 0)
        return carry

    lax.fori_loop(0, NCHUNK, chunk, 0)
    pltpu.sync_copy(uemb, out_hbm.at[pl.ds(base * D, PERW * D)])


@functools.partial(
    pl.kernel,
    out_type=jax.ShapeDtypeStruct((B * NV,), jnp.float32),
    mesh=_mesh,
    scratch_types=[
        pltpu.VMEM((PERW * NV,), jnp.int32),
        pltpu.VMEM((C * NV, D), jnp.float32),
        pltpu.VMEM((PERW * D,), jnp.float32),
        pltpu.VMEM((PERW * NV,), jnp.float32),
        pltpu.SemaphoreType.DMA,
    ],
    compiler_params=_SC_PARAMS,
)
def _sc_dots(vidx_hbm, uemb_hbm, v_tab, out_hbm,
             vidx_v, vrows, uemb, scores, sem):
    wid = lax.axis_index("s") * NC + lax.axis_index("c")
    base = wid * PERW
    lane0 = lax.iota(jnp.int32, 16) == 0
    pltpu.sync_copy(vidx_hbm.at[pl.ds(base * NV, PERW * NV)], vidx_v)
    pltpu.sync_copy(uemb_hbm.at[pl.ds(base * D, PERW * D)], uemb)

    def chunk(ch, carry):
        handles = []
        for off, n in V_GROUPS:
            handles.append(pltpu.async_copy(
                v_tab.at[vidx_v.at[pl.ds(ch * (C * NV) + off, n)]],
                vrows.at[pl.ds(off, n)], sem))
        for h in handles:
            h.wait()

        def elem(e, carry2):
            ebase = (ch * C + e) * D
            acc = [uemb[pl.ds(ebase + k * 16, 16)] for k in range(4)]
            vrow0 = e * NV
            sbase = (ch * C + e) * NV
            for t in range(NV):
                prods = [vrows[vrow0 + t, pl.ds(k * 16, 16)] * acc[k]
                         for k in range(4)]
                s = (prods[0] + prods[1]) + (prods[2] + prods[3])
                dot = jnp.sum(s)
                dot = dot if t == 0 else -dot
                plsc.store_scatter(
                    scores,
                    [jnp.full((16,), sbase + t, dtype=jnp.int32)],
                    jnp.full((16,), dot, dtype=jnp.float32),
                    mask=lane0)
            return carry2

        lax.fori_loop(0, C, elem, 0)
        return carry

    lax.fori_loop(0, NCHUNK, chunk, 0)
    pltpu.sync_copy(scores, out_hbm.at[pl.ds(base * NV, PERW * NV)])


def _loss_body(x_ref, o_ref):
    o_ref[0, 0] = -jnp.sum(jax.nn.log_sigmoid(x_ref[...]))


_loss = pl.pallas_call(
    _loss_body,
    out_shape=jax.ShapeDtypeStruct((1, 1), jnp.float32),
    out_specs=pl.BlockSpec(memory_space=pltpu.SMEM),
)


def kernel(batch_0, batch_1, batch_2, u_table, v_table):
    uidx = batch_0.astype(jnp.int32).reshape(B * CTX)
    vidx = jnp.concatenate(
        [batch_1[:, None], batch_2], axis=1).astype(jnp.int32).reshape(B * NV)
    uemb = _sc_pool(uidx, u_table)
    scores = _sc_dots(vidx, uemb, v_table)
    loss = _loss(scores.reshape(B * NV // 128, 128))
    return loss.reshape(())


# speedup vs baseline: 1.5185x; 1.0131x over previous
import functools

import jax
import jax.numpy as jnp
from jax import lax
from jax.experimental import pallas as pl
from jax.experimental.pallas import tpu as pltpu
from jax.experimental.pallas import tpu_sc as plsc

B = 16384
D = 64
CTX = 10
NEG = 5
NV = 1 + NEG
NC = 2
NS = 16
NW = NC * NS
PERW = B // NW
C = 32
NCHUNK = PERW // C

U_GROUPS = [(0, 128), (128, 128), (256, 64)]
V_GROUPS = [(0, 128), (128, 64)]

_mesh = plsc.VectorSubcoreMesh(
    core_axis_name="c", subcore_axis_name="s", num_cores=NC, num_subcores=NS)

_SC_PARAMS = pltpu.CompilerParams(
    needs_layout_passes=False, use_tc_tiling_on_sc=False)


@functools.partial(
    pl.kernel,
    out_type=jax.ShapeDtypeStruct((B * D,), jnp.float32),
    mesh=_mesh,
    scratch_types=[
        pltpu.VMEM((PERW * CTX,), jnp.int32),
        pltpu.VMEM((2, C * CTX, D), jnp.float32),
        pltpu.VMEM((PERW * D,), jnp.float32),
        pltpu.SemaphoreType.DMA,
        pltpu.SemaphoreType.DMA,
    ],
    compiler_params=_SC_PARAMS,
)
def _sc_pool(uidx_hbm, u_tab, out_hbm, uidx_v, urows, uemb, sem0, sem1):
    wid = lax.axis_index("s") * NC + lax.axis_index("c")
    base = wid * PERW
    pltpu.sync_copy(uidx_hbm.at[pl.ds(base * CTX, PERW * CTX)], uidx_v)

    def issue(ch, par, sem):
        for off, n in U_GROUPS:
            pltpu.async_copy(
                u_tab.at[uidx_v.at[pl.ds(ch * (C * CTX) + off, n)]],
                urows.at[par, pl.ds(off, n)], sem)

    def drain(ch, par, sem):
        for off, n in U_GROUPS:
            pltpu.make_async_copy(
                u_tab.at[uidx_v.at[pl.ds(ch * (C * CTX) + off, n)]],
                urows.at[par, pl.ds(off, n)], sem).wait()

    def compute(ch, par):
        def elem(e, carry2):
            urow0 = e * CTX
            acc = [urows[par, urow0, pl.ds(k * 16, 16)] for k in range(4)]
            for c in range(1, CTX):
                for k in range(4):
                    acc[k] = acc[k] + urows[par, urow0 + c, pl.ds(k * 16, 16)]
            ebase = (ch * C + e) * D
            for k in range(4):
                uemb[pl.ds(ebase + k * 16, 16)] = acc[k] * (1.0 / CTX)
            return carry2

        lax.fori_loop(0, C, elem, 0)

    issue(0, 0, sem0)
    issue(1, 1, sem1)

    def pair(i, carry):
        ch0 = i * 2
        drain(ch0, 0, sem0)

        @pl.when(ch0 + 2 < NCHUNK)
        def _n0():
            issue(ch0 + 2, 0, sem0)

        compute(ch0, 0)
        drain(ch0 + 1, 1, sem1)

        @pl.when(ch0 + 3 < NCHUNK)
        def _n1():
            issue(ch0 + 3, 1, sem1)

        compute(ch0 + 1, 1)
        return carry

    lax.fori_loop(0, NCHUNK // 2, pair, 0)
    pltpu.sync_copy(uemb, out_hbm.at[pl.ds(base * D, PERW * D)])


@functools.partial(
    pl.kernel,
    out_type=jax.ShapeDtypeStruct((B * NV,), jnp.float32),
    mesh=_mesh,
    scratch_types=[
        pltpu.VMEM((PERW * NV,), jnp.int32),
        pltpu.VMEM((2, C * NV, D), jnp.float32),
        pltpu.VMEM((PERW * D,), jnp.float32),
        pltpu.VMEM((PERW * NV,), jnp.float32),
        pltpu.SemaphoreType.DMA,
        pltpu.SemaphoreType.DMA,
    ],
    compiler_params=_SC_PARAMS,
)
def _sc_dots(vidx_hbm, uemb_hbm, v_tab, out_hbm,
             vidx_v, vrows, uemb, scores, sem0, sem1):
    wid = lax.axis_index("s") * NC + lax.axis_index("c")
    base = wid * PERW
    lane0 = lax.iota(jnp.int32, 16) == 0
    pltpu.sync_copy(vidx_hbm.at[pl.ds(base * NV, PERW * NV)], vidx_v)
    pltpu.sync_copy(uemb_hbm.at[pl.ds(base * D, PERW * D)], uemb)

    def issue(ch, par, sem):
        for off, n in V_GROUPS:
            pltpu.async_copy(
                v_tab.at[vidx_v.at[pl.ds(ch * (C * NV) + off, n)]],
                vrows.at[par, pl.ds(off, n)], sem)

    def drain(ch, par, sem):
        for off, n in V_GROUPS:
            pltpu.make_async_copy(
                v_tab.at[vidx_v.at[pl.ds(ch * (C * NV) + off, n)]],
                vrows.at[par, pl.ds(off, n)], sem).wait()

    def compute(ch, par):
        def elem(e, carry2):
            ebase = (ch * C + e) * D
            acc = [uemb[pl.ds(ebase + k * 16, 16)] for k in range(4)]
            vrow0 = e * NV
            sbase = (ch * C + e) * NV
            for t in range(NV):
                prods = [vrows[par, vrow0 + t, pl.ds(k * 16, 16)] * acc[k]
                         for k in range(4)]
                s = (prods[0] + prods[1]) + (prods[2] + prods[3])
                dot = jnp.sum(s)
                dot = dot if t == 0 else -dot
                plsc.store_scatter(
                    scores,
                    [jnp.full((16,), sbase + t, dtype=jnp.int32)],
                    jnp.full((16,), dot, dtype=jnp.float32),
                    mask=lane0)
            return carry2

        lax.fori_loop(0, C, elem, 0)

    issue(0, 0, sem0)
    issue(1, 1, sem1)

    def pair(i, carry):
        ch0 = i * 2
        drain(ch0, 0, sem0)

        @pl.when(ch0 + 2 < NCHUNK)
        def _n0():
            issue(ch0 + 2, 0, sem0)

        compute(ch0, 0)
        drain(ch0 + 1, 1, sem1)

        @pl.when(ch0 + 3 < NCHUNK)
        def _n1():
            issue(ch0 + 3, 1, sem1)

        compute(ch0 + 1, 1)
        return carry

    lax.fori_loop(0, NCHUNK // 2, pair, 0)
    pltpu.sync_copy(scores, out_hbm.at[pl.ds(base * NV, PERW * NV)])


def _loss_body(x_ref, o_ref):
    o_ref[0, 0] = -jnp.sum(jax.nn.log_sigmoid(x_ref[...]))


_loss = pl.pallas_call(
    _loss_body,
    out_shape=jax.ShapeDtypeStruct((1, 1), jnp.float32),
    out_specs=pl.BlockSpec(memory_space=pltpu.SMEM),
)


def kernel(batch_0, batch_1, batch_2, u_table, v_table):
    uidx = batch_0.astype(jnp.int32).reshape(B * CTX)
    vidx = jnp.concatenate(
        [batch_1[:, None], batch_2], axis=1).astype(jnp.int32).reshape(B * NV)
    uemb = _sc_pool(uidx, u_table)
    scores = _sc_dots(vidx, uemb, v_table)
    loss = _loss(scores.reshape(B * NV // 128, 128))
    return loss.reshape(())
